# baseline (speedup 1.0000x reference)
import functools

import jax
import jax.numpy as jnp
from jax import lax
from jax.experimental import pallas as pl
from jax.experimental.pallas import tpu as pltpu
from jax.experimental.pallas import tpu_sc as plsc

N = 10000
NP = 10240
E = 160000
D = 256
H = D // 2
NC = 2
NS = 16
CH = 100
CHS = 128

EP = NS * 80 * CHS
EPS_SEG = EP // NS
NCHUNK_SEG = EPS_SEG // CHS
EPS_HIST = E // (NC * NS)
NCHUNK_HIST = EPS_HIST // CH
KH = 5
RPS = NP // NS

_F32 = jnp.float32


def _mesh():
    return plsc.VectorSubcoreMesh(
        core_axis_name="c", subcore_axis_name="s", num_cores=NC, num_subcores=NS
    )


@functools.cache
def _sc_degree_hist_fn():
    @functools.partial(
        pl.kernel,
        out_type=jax.ShapeDtypeStruct((NC, NP, H), _F32),
        mesh=_mesh(),
        scratch_types=[
            pltpu.VMEM((NCHUNK_HIST, CH), jnp.int32),
            pltpu.VMEM((CH, H), _F32),
            pltpu.VMEM_SHARED((NP, H), _F32),
            [pltpu.SemaphoreType.DMA for _ in range(KH)],
        ],
    )
    def _sc_degree_hist(dst_hbm, ones_hbm, zeros_hbm, out_hbm, idx_v, ones_v, acc,
                        hsem):
        c = lax.axis_index("c")
        s = lax.axis_index("s")
        pltpu.sync_copy(zeros_hbm.at[pl.ds(s * RPS, RPS)], acc.at[pl.ds(s * RPS, RPS)])
        pltpu.sync_copy(ones_hbm, ones_v)
        pltpu.sync_copy(dst_hbm.at[c, s], idx_v)
        plsc.subcore_barrier()

        for k in range(KH):
            pltpu.async_copy(ones_v, acc.at[idx_v.at[k]], hsem[k], add=True)

        @pl.loop(0, NCHUNK_HIST // KH - 1)
        def _(r):
            for k in range(KH):
                pltpu.make_async_copy(ones_v, acc.at[idx_v.at[0]], hsem[k]).wait()
                pltpu.async_copy(ones_v, acc.at[idx_v.at[(r + 1) * KH + k]],
                                 hsem[k], add=True)

        for k in range(KH):
            pltpu.make_async_copy(ones_v, acc.at[idx_v.at[0]], hsem[k]).wait()

        plsc.subcore_barrier()
        pltpu.sync_copy(acc.at[pl.ds(s * RPS, RPS)], out_hbm.at[c, pl.ds(s * RPS, RPS)])

    return _sc_degree_hist


NB = 2
NPHASE = 2
NHALF = NCHUNK_SEG // NPHASE
NFULL = NHALF // NB
NREM = NHALF - NFULL * NB


@functools.cache
def _sc_segsum_fn():
    @functools.partial(
        pl.kernel,
        out_type=[
            jax.ShapeDtypeStruct((NP, H), _F32),
            jax.ShapeDtypeStruct((NP, H), _F32),
        ],
        mesh=_mesh(),
        scratch_types=[
            pltpu.VMEM((NHALF, CHS), jnp.int32),
            pltpu.VMEM((NHALF, CHS), jnp.int32),
            [pltpu.VMEM((CHS, H), _F32) for _ in range(NB)],
            pltpu.VMEM_SHARED((NP, H), _F32),
            [pltpu.SemaphoreType.DMA for _ in range(NB)],
            [pltpu.SemaphoreType.DMA for _ in range(NB)],
        ],
    )
    def _sc_segsum(ga_hbm, gb_hbm, src_hbm, dst_hbm, oa_hbm, ob_hbm,
                   src_v, dst_v, bufs, acc, gsem, ssem):
        c = lax.axis_index("c")
        s = lax.axis_index("s")

        def half(g_hbm, o_hbm):
            def start_gather(j, b):
                pltpu.async_copy(g_hbm.at[src_v.at[j]], bufs[b], gsem[b])

            def wait_gather(b):
                pltpu.make_async_copy(g_hbm.at[src_v.at[0]], bufs[b], gsem[b]).wait()

            def start_scatter(j, b):
                pltpu.async_copy(bufs[b], acc.at[dst_v.at[j]], ssem[b], add=True)

            def wait_scatter(b):
                pltpu.make_async_copy(bufs[b], acc.at[dst_v.at[0]], ssem[b]).wait()

            for ph in range(NPHASE):
                pltpu.sync_copy(src_hbm.at[s, ph], src_v)
                pltpu.sync_copy(dst_hbm.at[s, ph], dst_v)
                for b in range(NB):
                    start_gather(b, b)
                if ph == 0:
                    pltpu.sync_copy(g_hbm.at[pl.ds(s * RPS, RPS)],
                                    acc.at[pl.ds(s * RPS, RPS)])
                    plsc.subcore_barrier()

                @pl.loop(0, NFULL - 1)
                def _(r):
                    for b in range(NB):
                        wait_gather(b)
                        start_scatter(r * NB + b, b)
                    for b in range(NB):
                        wait_scatter(b)
                        start_gather((r + 1) * NB + b, b)

                for b in range(NB):
                    wait_gather(b)
                    start_scatter((NFULL - 1) * NB + b, b)
                for k in range(NREM):
                    wait_scatter(k)
                    start_gather(NFULL * NB + k, k)
                for k in range(NREM):
                    wait_gather(k)
                    start_scatter(NFULL * NB + k, k)
                for b in range(NB):
                    wait_scatter(b)

            plsc.subcore_barrier()
            pltpu.sync_copy(acc.at[pl.ds(s * RPS, RPS)], o_hbm.at[pl.ds(s * RPS, RPS)])

        @pl.when(c == 0)
        def _():
            half(ga_hbm, oa_hbm)

        @pl.when(c == 1)
        def _():
            half(gb_hbm, ob_hbm)

    return _sc_segsum


RB = 1024


def _dinv_block(hist_blk):
    deg = hist_blk[0, :, 0] + hist_blk[1, :, 0] + 1.0
    return lax.rsqrt(deg)[:, None]


def _mm2_body(x_ref, w0_ref, wr_ref, hw_ref, x0_ref):
    xb = x_ref[...]
    hw_ref[...] = jnp.dot(xb, w0_ref[...], preferred_element_type=_F32)
    x0_ref[...] = jnp.dot(xb, wr_ref[...], preferred_element_type=_F32)


def _tc_mm2(x, w0, wres):
    return pl.pallas_call(
        _mm2_body,
        grid=(NP // RB,),
        in_specs=[
            pl.BlockSpec((RB, D), lambda i: (i, 0)),
            pl.BlockSpec((D, D), lambda i: (0, 0)),
            pl.BlockSpec((D, D), lambda i: (0, 0)),
        ],
        out_specs=[
            pl.BlockSpec((RB, D), lambda i: (i, 0)),
            pl.BlockSpec((RB, D), lambda i: (i, 0)),
        ],
        out_shape=[
            jax.ShapeDtypeStruct((NP, D), _F32),
            jax.ShapeDtypeStruct((NP, D), _F32),
        ],
    )(x, w0, wres)


def _scale_split_body(hw_ref, hist_ref, ga_ref, gb_ref, dinv_ref):
    dinv = _dinv_block(hist_ref[...])
    g = hw_ref[...] * dinv
    ga_ref[...] = g[:, :H]
    gb_ref[...] = g[:, H:]
    dinv_ref[...] = jnp.broadcast_to(dinv, (RB, 128))


def _tc_scale_split(hw, hist):
    return pl.pallas_call(
        _scale_split_body,
        grid=(NP // RB,),
        in_specs=[
            pl.BlockSpec((RB, D), lambda i: (i, 0)),
            pl.BlockSpec((NC, RB, 128), lambda i: (0, i, 0)),
        ],
        out_specs=[
            pl.BlockSpec((RB, H), lambda i: (i, 0)),
            pl.BlockSpec((RB, H), lambda i: (i, 0)),
            pl.BlockSpec((RB, 128), lambda i: (i, 0)),
        ],
        out_shape=[
            jax.ShapeDtypeStruct((NP, H), _F32),
            jax.ShapeDtypeStruct((NP, H), _F32),
            jax.ShapeDtypeStruct((NP, 128), _F32),
        ],
    )(hw, hist)


def _mid_body(sa_ref, sb_ref, x0_ref, dinv_ref, w_ref, ga_ref, gb_ref):
    dinv = dinv_ref[:, :1]
    t = jnp.concatenate([sa_ref[...], sb_ref[...]], axis=1) * dinv
    t = jnp.maximum(t, 0.0)
    nrm = jnp.sqrt(jnp.sum(t * t, axis=1, keepdims=True))
    t = t / jnp.maximum(nrm, 1e-12)
    h = t + x0_ref[...]
    g = jnp.dot(h, w_ref[...], preferred_element_type=_F32) * dinv
    ga_ref[...] = g[:, :H]
    gb_ref[...] = g[:, H:]


def _tc_mid(sa, sb, x0, dinv, w):
    return pl.pallas_call(
        _mid_body,
        grid=(NP // RB,),
        in_specs=[
            pl.BlockSpec((RB, H), lambda i: (i, 0)),
            pl.BlockSpec((RB, H), lambda i: (i, 0)),
            pl.BlockSpec((RB, D), lambda i: (i, 0)),
            pl.BlockSpec((RB, 128), lambda i: (i, 0)),
            pl.BlockSpec((D, D), lambda i: (0, 0)),
        ],
        out_specs=[
            pl.BlockSpec((RB, H), lambda i: (i, 0)),
            pl.BlockSpec((RB, H), lambda i: (i, 0)),
        ],
        out_shape=[
            jax.ShapeDtypeStruct((NP, H), _F32),
            jax.ShapeDtypeStruct((NP, H), _F32),
        ],
    )(sa, sb, x0, dinv, w)


def _final_body(sa_ref, sb_ref, dinv_ref, out_ref):
    dinv = dinv_ref[:, :1]
    out_ref[...] = jnp.concatenate([sa_ref[...], sb_ref[...]], axis=1) * dinv


def _tc_final(sa, sb, dinv):
    return pl.pallas_call(
        _final_body,
        grid=(NP // RB,),
        in_specs=[
            pl.BlockSpec((RB, H), lambda i: (i, 0)),
            pl.BlockSpec((RB, H), lambda i: (i, 0)),
            pl.BlockSpec((RB, 128), lambda i: (i, 0)),
        ],
        out_specs=pl.BlockSpec((RB, D), lambda i: (i, 0)),
        out_shape=jax.ShapeDtypeStruct((NP, D), _F32),
    )(sa, sb, dinv)


def kernel(x, edge_index, W0, W1, W2, W_res):
    src = edge_index[0].astype(jnp.int32)
    dst = edge_index[1].astype(jnp.int32)
    pad_e = jnp.full((EP - E,), N, jnp.int32)
    src_seg = jnp.concatenate([src, pad_e]).reshape(NS, NPHASE, NHALF, CHS)
    dst_seg = jnp.concatenate([dst, pad_e]).reshape(NS, NPHASE, NHALF, CHS)
    dst_hist = dst.reshape(NC, NS, NCHUNK_HIST, CH)
    ones128 = jnp.ones((CH, H), _F32)
    zeros128 = jnp.zeros((NP, H), _F32)
    x = jnp.pad(x, ((0, NP - N), (0, 0)))

    hist = _sc_degree_hist_fn()(dst_hist, ones128, zeros128)
    hw0, x0 = _tc_mm2(x, W0, W_res)
    ga, gb, dinv = _tc_scale_split(hw0, hist)
    for w in (W1, W2, None):
        sa, sb = _sc_segsum_fn()(ga, gb, src_seg, dst_seg)
        if w is None:
            return _tc_final(sa, sb, dinv)[:N]
        ga, gb = _tc_mid(sa, sb, x0, dinv, w)


# speedup vs baseline: 2.1211x; 2.1211x over previous
import functools

import jax
import jax.numpy as jnp
from jax import lax
from jax.experimental import pallas as pl
from jax.experimental.pallas import tpu as pltpu
from jax.experimental.pallas import tpu_sc as plsc

N = 10000
NP = 10240
E = 160000
D = 256
H = D // 2
NC = 2
NS = 16
CH = 100
CHS = 100

EPS_SEG = E // NS
NCHUNK_SEG = EPS_SEG // CHS
EPS_HIST = E // (NC * NS)
NCHUNK_HIST = EPS_HIST // CH
KH = 5
RPS = NP // NS

_F32 = jnp.float32


def _mesh():
    return plsc.VectorSubcoreMesh(
        core_axis_name="c", subcore_axis_name="s", num_cores=NC, num_subcores=NS
    )


@functools.cache
def _sc_degree_hist_fn():
    @functools.partial(
        pl.kernel,
        out_type=jax.ShapeDtypeStruct((NC, NP, H), _F32),
        mesh=_mesh(),
        scratch_types=[
            pltpu.VMEM((NCHUNK_HIST, CH), jnp.int32),
            pltpu.VMEM((CH, H), _F32),
            pltpu.VMEM_SHARED((NP, H), _F32),
            [pltpu.SemaphoreType.DMA for _ in range(KH)],
        ],
    )
    def _sc_degree_hist(dst_hbm, ones_hbm, zeros_hbm, out_hbm, idx_v, ones_v, acc,
                        hsem):
        c = lax.axis_index("c")
        s = lax.axis_index("s")
        pltpu.sync_copy(zeros_hbm.at[pl.ds(s * RPS, RPS)], acc.at[pl.ds(s * RPS, RPS)])
        pltpu.sync_copy(ones_hbm, ones_v)
        pltpu.sync_copy(dst_hbm.at[c, s], idx_v)
        plsc.subcore_barrier()

        for k in range(KH):
            pltpu.async_copy(ones_v, acc.at[idx_v.at[k]], hsem[k], add=True)

        @pl.loop(0, NCHUNK_HIST // KH - 1)
        def _(r):
            for k in range(KH):
                pltpu.make_async_copy(ones_v, acc.at[idx_v.at[0]], hsem[k]).wait()
                pltpu.async_copy(ones_v, acc.at[idx_v.at[(r + 1) * KH + k]],
                                 hsem[k], add=True)

        for k in range(KH):
            pltpu.make_async_copy(ones_v, acc.at[idx_v.at[0]], hsem[k]).wait()

        plsc.subcore_barrier()
        pltpu.sync_copy(acc.at[pl.ds(s * RPS, RPS)], out_hbm.at[c, pl.ds(s * RPS, RPS)])

    return _sc_degree_hist


NB = 3
NPHASE = 4
NHALF = NCHUNK_SEG // NPHASE
NFULL = NHALF // NB
NREM = NHALF - NFULL * NB


@functools.cache
def _sc_segsum_fn():
    @functools.partial(
        pl.kernel,
        out_type=[
            jax.ShapeDtypeStruct((NP, H), _F32),
            jax.ShapeDtypeStruct((NP, H), _F32),
        ],
        mesh=_mesh(),
        scratch_types=[
            pltpu.VMEM((NHALF, CHS), jnp.int32),
            pltpu.VMEM((NHALF, CHS), jnp.int32),
            [pltpu.VMEM((CHS, H), _F32) for _ in range(NB)],
            pltpu.VMEM_SHARED((NP, H), _F32),
            [pltpu.SemaphoreType.DMA for _ in range(NB)],
            [pltpu.SemaphoreType.DMA for _ in range(NB)],
        ],
    )
    def _sc_segsum(ga_hbm, gb_hbm, src_hbm, dst_hbm, oa_hbm, ob_hbm,
                   src_v, dst_v, bufs, acc, gsem, ssem):
        c = lax.axis_index("c")
        s = lax.axis_index("s")

        def half(g_hbm, o_hbm):
            def start_gather(j, b):
                pltpu.async_copy(g_hbm.at[src_v.at[j]], bufs[b], gsem[b])

            def wait_gather(b):
                pltpu.make_async_copy(g_hbm.at[src_v.at[0]], bufs[b], gsem[b]).wait()

            def start_scatter(j, b):
                pltpu.async_copy(bufs[b], acc.at[dst_v.at[j]], ssem[b], add=True)

            def wait_scatter(b):
                pltpu.make_async_copy(bufs[b], acc.at[dst_v.at[0]], ssem[b]).wait()

            for ph in range(NPHASE):
                pltpu.sync_copy(src_hbm.at[s, ph], src_v)
                pltpu.sync_copy(dst_hbm.at[s, ph], dst_v)
                for b in range(NB):
                    start_gather(b, b)
                if ph == 0:
                    pltpu.sync_copy(g_hbm.at[pl.ds(s * RPS, RPS)],
                                    acc.at[pl.ds(s * RPS, RPS)])
                    plsc.subcore_barrier()

                @pl.loop(0, NFULL - 1)
                def _(r):
                    for b in range(NB):
                        wait_gather(b)
                        start_scatter(r * NB + b, b)
                    for b in range(NB):
                        wait_scatter(b)
                        start_gather((r + 1) * NB + b, b)

                for b in range(NB):
                    wait_gather(b)
                    start_scatter((NFULL - 1) * NB + b, b)
                for k in range(NREM):
                    wait_scatter(k)
                    start_gather(NFULL * NB + k, k)
                for k in range(NREM):
                    wait_gather(k)
                    start_scatter(NFULL * NB + k, k)
                for b in range(NB):
                    wait_scatter(b)

            plsc.subcore_barrier()
            pltpu.sync_copy(acc.at[pl.ds(s * RPS, RPS)], o_hbm.at[pl.ds(s * RPS, RPS)])

        @pl.when(c == 0)
        def _():
            half(ga_hbm, oa_hbm)

        @pl.when(c == 1)
        def _():
            half(gb_hbm, ob_hbm)

    return _sc_segsum


RB = 1024


def _dinv_block(hist_blk):
    deg = hist_blk[0, :, 0] + hist_blk[1, :, 0] + 1.0
    return lax.rsqrt(deg)[:, None]


def _mm2_body(x_ref, w0_ref, wr_ref, hw_ref, x0_ref):
    xb = x_ref[...]
    hw_ref[...] = jnp.dot(xb, w0_ref[...], preferred_element_type=_F32)
    x0_ref[...] = jnp.dot(xb, wr_ref[...], preferred_element_type=_F32)


def _tc_mm2(x, w0, wres):
    return pl.pallas_call(
        _mm2_body,
        grid=(NP // RB,),
        in_specs=[
            pl.BlockSpec((RB, D), lambda i: (i, 0)),
            pl.BlockSpec((D, D), lambda i: (0, 0)),
            pl.BlockSpec((D, D), lambda i: (0, 0)),
        ],
        out_specs=[
            pl.BlockSpec((RB, D), lambda i: (i, 0)),
            pl.BlockSpec((RB, D), lambda i: (i, 0)),
        ],
        out_shape=[
            jax.ShapeDtypeStruct((NP, D), _F32),
            jax.ShapeDtypeStruct((NP, D), _F32),
        ],
    )(x, w0, wres)


def _scale_split_body(hw_ref, hist_ref, ga_ref, gb_ref, dinv_ref):
    dinv = _dinv_block(hist_ref[...])
    g = hw_ref[...] * dinv
    ga_ref[...] = g[:, :H]
    gb_ref[...] = g[:, H:]
    dinv_ref[...] = jnp.broadcast_to(dinv, (RB, 128))


def _tc_scale_split(hw, hist):
    return pl.pallas_call(
        _scale_split_body,
        grid=(NP // RB,),
        in_specs=[
            pl.BlockSpec((RB, D), lambda i: (i, 0)),
            pl.BlockSpec((NC, RB, 128), lambda i: (0, i, 0)),
        ],
        out_specs=[
            pl.BlockSpec((RB, H), lambda i: (i, 0)),
            pl.BlockSpec((RB, H), lambda i: (i, 0)),
            pl.BlockSpec((RB, 128), lambda i: (i, 0)),
        ],
        out_shape=[
            jax.ShapeDtypeStruct((NP, H), _F32),
            jax.ShapeDtypeStruct((NP, H), _F32),
            jax.ShapeDtypeStruct((NP, 128), _F32),
        ],
    )(hw, hist)


def _mid_body(sa_ref, sb_ref, x0_ref, dinv_ref, w_ref, ga_ref, gb_ref):
    dinv = dinv_ref[:, :1]
    t = jnp.concatenate([sa_ref[...], sb_ref[...]], axis=1) * dinv
    t = jnp.maximum(t, 0.0)
    nrm = jnp.sqrt(jnp.sum(t * t, axis=1, keepdims=True))
    t = t / jnp.maximum(nrm, 1e-12)
    h = t + x0_ref[...]
    g = jnp.dot(h, w_ref[...], preferred_element_type=_F32) * dinv
    ga_ref[...] = g[:, :H]
    gb_ref[...] = g[:, H:]


def _tc_mid(sa, sb, x0, dinv, w):
    return pl.pallas_call(
        _mid_body,
        grid=(NP // RB,),
        in_specs=[
            pl.BlockSpec((RB, H), lambda i: (i, 0)),
            pl.BlockSpec((RB, H), lambda i: (i, 0)),
            pl.BlockSpec((RB, D), lambda i: (i, 0)),
            pl.BlockSpec((RB, 128), lambda i: (i, 0)),
            pl.BlockSpec((D, D), lambda i: (0, 0)),
        ],
        out_specs=[
            pl.BlockSpec((RB, H), lambda i: (i, 0)),
            pl.BlockSpec((RB, H), lambda i: (i, 0)),
        ],
        out_shape=[
            jax.ShapeDtypeStruct((NP, H), _F32),
            jax.ShapeDtypeStruct((NP, H), _F32),
        ],
    )(sa, sb, x0, dinv, w)


def _final_body(sa_ref, sb_ref, dinv_ref, out_ref):
    dinv = dinv_ref[:, :1]
    out_ref[...] = jnp.concatenate([sa_ref[...], sb_ref[...]], axis=1) * dinv


def _tc_final(sa, sb, dinv):
    return pl.pallas_call(
        _final_body,
        grid=(NP // RB,),
        in_specs=[
            pl.BlockSpec((RB, H), lambda i: (i, 0)),
            pl.BlockSpec((RB, H), lambda i: (i, 0)),
            pl.BlockSpec((RB, 128), lambda i: (i, 0)),
        ],
        out_specs=pl.BlockSpec((RB, D), lambda i: (i, 0)),
        out_shape=jax.ShapeDtypeStruct((NP, D), _F32),
    )(sa, sb, dinv)


def kernel(x, edge_index, W0, W1, W2, W_res):
    src = edge_index[0].astype(jnp.int32)
    dst = edge_index[1].astype(jnp.int32)
    src_seg = src.reshape(NS, NPHASE, NHALF, CHS)
    dst_seg = dst.reshape(NS, NPHASE, NHALF, CHS)
    dst_hist = dst.reshape(NC, NS, NCHUNK_HIST, CH)
    ones128 = jnp.ones((CH, H), _F32)
    zeros128 = jnp.zeros((NP, H), _F32)
    x = jnp.pad(x, ((0, NP - N), (0, 0)))

    hist = _sc_degree_hist_fn()(dst_hist, ones128, zeros128)
    hw0, x0 = _tc_mm2(x, W0, W_res)
    ga, gb, dinv = _tc_scale_split(hw0, hist)
    for w in (W1, W2, None):
        sa, sb = _sc_segsum_fn()(ga, gb, src_seg, dst_seg)
        if w is None:
            return _tc_final(sa, sb, dinv)[:N]
        ga, gb = _tc_mid(sa, sb, x0, dinv, w)


# speedup vs baseline: 2.1847x; 1.0300x over previous
import functools

import jax
import jax.numpy as jnp
from jax import lax
from jax.experimental import pallas as pl
from jax.experimental.pallas import tpu as pltpu
from jax.experimental.pallas import tpu_sc as plsc

N = 10000
NP = 10240
E = 160000
D = 256
H = D // 2
NC = 2
NS = 16
CH = 100
CHS = 50

EPS_SEG = E // NS
NCHUNK_SEG = EPS_SEG // CHS
EPS_HIST = E // (NC * NS)
NCHUNK_HIST = EPS_HIST // CH
KH = 5
RPS = NP // NS

_F32 = jnp.float32


def _mesh():
    return plsc.VectorSubcoreMesh(
        core_axis_name="c", subcore_axis_name="s", num_cores=NC, num_subcores=NS
    )


@functools.cache
def _sc_degree_hist_fn():
    @functools.partial(
        pl.kernel,
        out_type=jax.ShapeDtypeStruct((NC, NP, H), _F32),
        mesh=_mesh(),
        scratch_types=[
            pltpu.VMEM((NCHUNK_HIST, CH), jnp.int32),
            pltpu.VMEM((CH, H), _F32),
            pltpu.VMEM_SHARED((NP, H), _F32),
            [pltpu.SemaphoreType.DMA for _ in range(KH)],
        ],
    )
    def _sc_degree_hist(dst_hbm, ones_hbm, zeros_hbm, out_hbm, idx_v, ones_v, acc,
                        hsem):
        c = lax.axis_index("c")
        s = lax.axis_index("s")
        pltpu.sync_copy(zeros_hbm.at[pl.ds(s * RPS, RPS)], acc.at[pl.ds(s * RPS, RPS)])
        pltpu.sync_copy(ones_hbm, ones_v)
        pltpu.sync_copy(dst_hbm.at[c, s], idx_v)
        plsc.subcore_barrier()

        for k in range(KH):
            pltpu.async_copy(ones_v, acc.at[idx_v.at[k]], hsem[k], add=True)

        @pl.loop(0, NCHUNK_HIST // KH - 1)
        def _(r):
            for k in range(KH):
                pltpu.make_async_copy(ones_v, acc.at[idx_v.at[0]], hsem[k]).wait()
                pltpu.async_copy(ones_v, acc.at[idx_v.at[(r + 1) * KH + k]],
                                 hsem[k], add=True)

        for k in range(KH):
            pltpu.make_async_copy(ones_v, acc.at[idx_v.at[0]], hsem[k]).wait()

        plsc.subcore_barrier()
        pltpu.sync_copy(acc.at[pl.ds(s * RPS, RPS)], out_hbm.at[c, pl.ds(s * RPS, RPS)])

    return _sc_degree_hist


NB = 4
NPHASE = 4
NHALF = NCHUNK_SEG // NPHASE
NFULL = NHALF // NB
NREM = NHALF - NFULL * NB


@functools.cache
def _sc_segsum_fn():
    @functools.partial(
        pl.kernel,
        out_type=[
            jax.ShapeDtypeStruct((NP, H), _F32),
            jax.ShapeDtypeStruct((NP, H), _F32),
        ],
        mesh=_mesh(),
        scratch_types=[
            pltpu.VMEM((NHALF, CHS), jnp.int32),
            pltpu.VMEM((NHALF, CHS), jnp.int32),
            [pltpu.VMEM((CHS, H), _F32) for _ in range(NB)],
            pltpu.VMEM_SHARED((NP, H), _F32),
            [pltpu.SemaphoreType.DMA for _ in range(NB)],
            [pltpu.SemaphoreType.DMA for _ in range(NB)],
        ],
    )
    def _sc_segsum(ga_hbm, gb_hbm, src_hbm, dst_hbm, oa_hbm, ob_hbm,
                   src_v, dst_v, bufs, acc, gsem, ssem):
        c = lax.axis_index("c")
        s = lax.axis_index("s")

        def half(g_hbm, o_hbm):
            def start_gather(j, b):
                pltpu.async_copy(g_hbm.at[src_v.at[j]], bufs[b], gsem[b])

            def wait_gather(b):
                pltpu.make_async_copy(g_hbm.at[src_v.at[0]], bufs[b], gsem[b]).wait()

            def start_scatter(j, b):
                pltpu.async_copy(bufs[b], acc.at[dst_v.at[j]], ssem[b], add=True)

            def wait_scatter(b):
                pltpu.make_async_copy(bufs[b], acc.at[dst_v.at[0]], ssem[b]).wait()

            for ph in range(NPHASE):
                pltpu.sync_copy(src_hbm.at[s, ph], src_v)
                pltpu.sync_copy(dst_hbm.at[s, ph], dst_v)
                for b in range(NB):
                    start_gather(b, b)
                if ph == 0:
                    pltpu.sync_copy(g_hbm.at[pl.ds(s * RPS, RPS)],
                                    acc.at[pl.ds(s * RPS, RPS)])
                    plsc.subcore_barrier()

                @pl.loop(0, NFULL - 1)
                def _(r):
                    for b in range(NB):
                        wait_gather(b)
                        start_scatter(r * NB + b, b)
                    for b in range(NB):
                        wait_scatter(b)
                        start_gather((r + 1) * NB + b, b)

                for b in range(NB):
                    wait_gather(b)
                    start_scatter((NFULL - 1) * NB + b, b)
                for k in range(NREM):
                    wait_scatter(k)
                    start_gather(NFULL * NB + k, k)
                for k in range(NREM):
                    wait_gather(k)
                    start_scatter(NFULL * NB + k, k)
                for b in range(NB):
                    wait_scatter(b)

            plsc.subcore_barrier()
            pltpu.sync_copy(acc.at[pl.ds(s * RPS, RPS)], o_hbm.at[pl.ds(s * RPS, RPS)])

        @pl.when(c == 0)
        def _():
            half(ga_hbm, oa_hbm)

        @pl.when(c == 1)
        def _():
            half(gb_hbm, ob_hbm)

    return _sc_segsum


RB = 1024


def _dinv_block(hist_blk):
    deg = hist_blk[0, :, 0] + hist_blk[1, :, 0] + 1.0
    return lax.rsqrt(deg)[:, None]


def _mm2_body(x_ref, w0_ref, wr_ref, hw_ref, x0_ref):
    xb = x_ref[...]
    hw_ref[...] = jnp.dot(xb, w0_ref[...], preferred_element_type=_F32)
    x0_ref[...] = jnp.dot(xb, wr_ref[...], preferred_element_type=_F32)


def _tc_mm2(x, w0, wres):
    return pl.pallas_call(
        _mm2_body,
        grid=(NP // RB,),
        in_specs=[
            pl.BlockSpec((RB, D), lambda i: (i, 0)),
            pl.BlockSpec((D, D), lambda i: (0, 0)),
            pl.BlockSpec((D, D), lambda i: (0, 0)),
        ],
        out_specs=[
            pl.BlockSpec((RB, D), lambda i: (i, 0)),
            pl.BlockSpec((RB, D), lambda i: (i, 0)),
        ],
        out_shape=[
            jax.ShapeDtypeStruct((NP, D), _F32),
            jax.ShapeDtypeStruct((NP, D), _F32),
        ],
    )(x, w0, wres)


def _scale_split_body(hw_ref, hist_ref, ga_ref, gb_ref, dinv_ref):
    dinv = _dinv_block(hist_ref[...])
    g = hw_ref[...] * dinv
    ga_ref[...] = g[:, :H]
    gb_ref[...] = g[:, H:]
    dinv_ref[...] = jnp.broadcast_to(dinv, (RB, 128))


def _tc_scale_split(hw, hist):
    return pl.pallas_call(
        _scale_split_body,
        grid=(NP // RB,),
        in_specs=[
            pl.BlockSpec((RB, D), lambda i: (i, 0)),
            pl.BlockSpec((NC, RB, 128), lambda i: (0, i, 0)),
        ],
        out_specs=[
            pl.BlockSpec((RB, H), lambda i: (i, 0)),
            pl.BlockSpec((RB, H), lambda i: (i, 0)),
            pl.BlockSpec((RB, 128), lambda i: (i, 0)),
        ],
        out_shape=[
            jax.ShapeDtypeStruct((NP, H), _F32),
            jax.ShapeDtypeStruct((NP, H), _F32),
            jax.ShapeDtypeStruct((NP, 128), _F32),
        ],
    )(hw, hist)


def _mid_body(sa_ref, sb_ref, x0_ref, dinv_ref, w_ref, ga_ref, gb_ref):
    dinv = dinv_ref[:, :1]
    t = jnp.concatenate([sa_ref[...], sb_ref[...]], axis=1) * dinv
    t = jnp.maximum(t, 0.0)
    nrm = jnp.sqrt(jnp.sum(t * t, axis=1, keepdims=True))
    t = t / jnp.maximum(nrm, 1e-12)
    h = t + x0_ref[...]
    g = jnp.dot(h, w_ref[...], preferred_element_type=_F32) * dinv
    ga_ref[...] = g[:, :H]
    gb_ref[...] = g[:, H:]


def _tc_mid(sa, sb, x0, dinv, w):
    return pl.pallas_call(
        _mid_body,
        grid=(NP // RB,),
        in_specs=[
            pl.BlockSpec((RB, H), lambda i: (i, 0)),
            pl.BlockSpec((RB, H), lambda i: (i, 0)),
            pl.BlockSpec((RB, D), lambda i: (i, 0)),
            pl.BlockSpec((RB, 128), lambda i: (i, 0)),
            pl.BlockSpec((D, D), lambda i: (0, 0)),
        ],
        out_specs=[
            pl.BlockSpec((RB, H), lambda i: (i, 0)),
            pl.BlockSpec((RB, H), lambda i: (i, 0)),
        ],
        out_shape=[
            jax.ShapeDtypeStruct((NP, H), _F32),
            jax.ShapeDtypeStruct((NP, H), _F32),
        ],
    )(sa, sb, x0, dinv, w)


def _final_body(sa_ref, sb_ref, dinv_ref, out_ref):
    dinv = dinv_ref[:, :1]
    out_ref[...] = jnp.concatenate([sa_ref[...], sb_ref[...]], axis=1) * dinv


def _tc_final(sa, sb, dinv):
    return pl.pallas_call(
        _final_body,
        grid=(NP // RB,),
        in_specs=[
            pl.BlockSpec((RB, H), lambda i: (i, 0)),
            pl.BlockSpec((RB, H), lambda i: (i, 0)),
            pl.BlockSpec((RB, 128), lambda i: (i, 0)),
        ],
        out_specs=pl.BlockSpec((RB, D), lambda i: (i, 0)),
        out_shape=jax.ShapeDtypeStruct((NP, D), _F32),
    )(sa, sb, dinv)


def kernel(x, edge_index, W0, W1, W2, W_res):
    src = edge_index[0].astype(jnp.int32)
    dst = edge_index[1].astype(jnp.int32)
    src_seg = src.reshape(NS, NPHASE, NHALF, CHS)
    dst_seg = dst.reshape(NS, NPHASE, NHALF, CHS)
    dst_hist = dst.reshape(NC, NS, NCHUNK_HIST, CH)
    ones128 = jnp.ones((CH, H), _F32)
    zeros128 = jnp.zeros((NP, H), _F32)
    x = jnp.pad(x, ((0, NP - N), (0, 0)))

    hist = _sc_degree_hist_fn()(dst_hist, ones128, zeros128)
    hw0, x0 = _tc_mm2(x, W0, W_res)
    ga, gb, dinv = _tc_scale_split(hw0, hist)
    for w in (W1, W2, None):
        sa, sb = _sc_segsum_fn()(ga, gb, src_seg, dst_seg)
        if w is None:
            return _tc_final(sa, sb, dinv)[:N]
        ga, gb = _tc_mid(sa, sb, x0, dinv, w)
